# baseline (speedup 1.0000x reference)
import jax
import jax.numpy as jnp
from jax.experimental import pallas as pl


def _body(t_ref, o_ref):
    o_ref[...] = jnp.broadcast_to(t_ref[...][None], o_ref.shape)


def kernel(x, pos_table):
    B = x.shape[0]
    P, D = pos_table.shape
    BLK = 512
    return pl.pallas_call(
        _body,
        grid=(P // BLK,),
        in_specs=[pl.BlockSpec((BLK, D), lambda i: (i, 0))],
        out_specs=pl.BlockSpec((B, BLK, D), lambda i: (0, i, 0)),
        out_shape=jax.ShapeDtypeStruct((B, P, D), jnp.float32),
    )(pos_table)
